# Initial kernel scaffold; baseline (speedup 1.0000x reference)
import jax
import jax.numpy as jnp
from jax.experimental import pallas as pl


def kernel(x, edge_index, W_self, b_self, W_s2d, b_s2d, W_d2s, b_d2s):
    raise NotImplementedError("write your pallas kernel here")


# speedup vs baseline: 6.2505x; 6.2505x over previous
import jax
import jax.numpy as jnp
from jax import lax
from jax.experimental import pallas as pl
from jax.experimental.pallas import tpu as pltpu
from jax.experimental.pallas import tpu_sc as plsc

N = 10000
E = 320000
D = 128
NUM_CORES = 2
NUM_SUBCORES = 16
EDGES_PER_TILE = E // NUM_SUBCORES
CHUNK = 128
FULL_CHUNKS = EDGES_PER_TILE // CHUNK
REM = EDGES_PER_TILE - FULL_CHUNKS * CHUNK
NPAD = 10240
STRIPE = NPAD // NUM_SUBCORES
C_S2D = 0.5
C_D2S = 0.5


def _sc_body(x_hbm, edge_hbm, s_out, deg_out,
             gidx_v, sidx_v, rows_v, ones_v, dstage_v,
             gidx_r, sidx_r, rows_r, accum, degacc, gsem):
    c = lax.axis_index("c")
    s = lax.axis_index("s")
    g_base = c * E
    s_base = (1 - c) * E

    zero16 = jnp.zeros((16,), jnp.float32)
    one16 = jnp.ones((16,), jnp.float32)

    def _zero_row(i, carry):
        for j in range(D // 16):
            rows_v[i, pl.ds(j * 16, 16)] = zero16
        return carry

    def _zero_dstage(i, carry):
        dstage_v[pl.ds(i * 16, 16)] = zero16
        return carry

    def _fill_ones(i, carry):
        ones_v[pl.ds(i * 16, 16)] = one16
        return carry

    lax.fori_loop(0, CHUNK, _zero_row, 0)
    lax.fori_loop(0, STRIPE // 16, _zero_dstage, 0)
    lax.fori_loop(0, CHUNK // 16, _fill_ones, 0)

    r0 = pl.multiple_of(s * STRIPE, 8)
    for k in range(STRIPE // CHUNK):
        pltpu.sync_copy(rows_v, accum.at[pl.ds(r0 + k * CHUNK, CHUNK)])
    pltpu.sync_copy(dstage_v, degacc.at[pl.ds(r0, STRIPE)])
    plsc.subcore_barrier()

    base = s * EDGES_PER_TILE

    def _chunk(g, carry):
        off = base + g * CHUNK
        pltpu.sync_copy(edge_hbm.at[pl.ds(pl.multiple_of(g_base + off, 8),
                                          CHUNK)], gidx_v)
        pltpu.sync_copy(edge_hbm.at[pl.ds(pl.multiple_of(s_base + off, 8),
                                          CHUNK)], sidx_v)
        pltpu.async_copy(x_hbm.at[gidx_v], rows_v, gsem).wait()
        pltpu.sync_copy(rows_v, accum.at[sidx_v], add=True)
        pltpu.sync_copy(ones_v, degacc.at[sidx_v], add=True)
        return carry

    lax.fori_loop(0, FULL_CHUNKS, _chunk, 0)

    offr = base + FULL_CHUNKS * CHUNK
    pltpu.sync_copy(edge_hbm.at[pl.ds(pl.multiple_of(g_base + offr, 8), REM)],
                    gidx_r)
    pltpu.sync_copy(edge_hbm.at[pl.ds(pl.multiple_of(s_base + offr, 8), REM)],
                    sidx_r)
    pltpu.async_copy(x_hbm.at[gidx_r], rows_r, gsem).wait()
    pltpu.sync_copy(rows_r, accum.at[sidx_r], add=True)
    pltpu.sync_copy(ones_v.at[pl.ds(0, REM)], degacc.at[sidx_r], add=True)

    plsc.subcore_barrier()

    for k in range(STRIPE // CHUNK):
        rk = pl.multiple_of(r0 + k * CHUNK, 8)
        pltpu.sync_copy(accum.at[pl.ds(rk, CHUNK)], rows_v)
        pltpu.sync_copy(rows_v, s_out.at[c, pl.ds(rk, CHUNK)])
    pltpu.sync_copy(degacc.at[pl.ds(r0, STRIPE)], dstage_v)
    pltpu.sync_copy(dstage_v,
                    deg_out.at[pl.ds(pl.multiple_of(c * NPAD + r0, 8),
                                     STRIPE)])


_sc_aggregate = pl.kernel(
    _sc_body,
    out_type=(
        jax.ShapeDtypeStruct((NUM_CORES, NPAD, D), jnp.float32),
        jax.ShapeDtypeStruct((NUM_CORES * NPAD,), jnp.float32),
    ),
    mesh=plsc.VectorSubcoreMesh(
        core_axis_name="c", subcore_axis_name="s",
        num_cores=NUM_CORES, num_subcores=NUM_SUBCORES),
    scratch_types=[
        pltpu.VMEM((CHUNK,), jnp.int32),
        pltpu.VMEM((CHUNK,), jnp.int32),
        pltpu.VMEM((CHUNK, D), jnp.float32),
        pltpu.VMEM((CHUNK,), jnp.float32),
        pltpu.VMEM((STRIPE,), jnp.float32),
        pltpu.VMEM((REM,), jnp.int32),
        pltpu.VMEM((REM,), jnp.int32),
        pltpu.VMEM((REM, D), jnp.float32),
        pltpu.VMEM_SHARED((NPAD, D), jnp.float32),
        pltpu.VMEM_SHARED((NPAD,), jnp.float32),
        pltpu.SemaphoreType.DMA,
    ],
)


BLK = 1000


def _fin_body(x_ref, s0_ref, s1_ref, d0_ref, d1_ref, ws_ref, w1_ref, w2_ref,
              bs_ref, b1_ref, b2_ref, o_ref):
    inv0 = C_S2D / jnp.maximum(d0_ref[...], 1.0)
    inv1 = C_D2S / jnp.maximum(d1_ref[...], 1.0)
    acc = jnp.dot(x_ref[...], ws_ref[...], preferred_element_type=jnp.float32)
    acc = acc + jnp.dot(s0_ref[...] * inv0, w1_ref[...],
                        preferred_element_type=jnp.float32)
    acc = acc + jnp.dot(s1_ref[...] * inv1, w2_ref[...],
                        preferred_element_type=jnp.float32)
    bias = bs_ref[...] + C_S2D * b1_ref[...] + C_D2S * b2_ref[...]
    o_ref[...] = acc + bias[None, :]


def _finalize(x, s0, s1, d0, d1, w_self, w_s2d, w_d2s, b_self, b_s2d, b_d2s):
    row_spec = pl.BlockSpec((BLK, D), lambda i: (i, 0))
    deg_spec = pl.BlockSpec((BLK, 1), lambda i: (i, 0))
    w_spec = pl.BlockSpec((D, D), lambda i: (0, 0))
    b_spec = pl.BlockSpec((D,), lambda i: (0,))
    return pl.pallas_call(
        _fin_body,
        grid=(N // BLK,),
        in_specs=[row_spec, row_spec, row_spec, deg_spec, deg_spec,
                  w_spec, w_spec, w_spec, b_spec, b_spec, b_spec],
        out_specs=row_spec,
        out_shape=jax.ShapeDtypeStruct((N, D), jnp.float32),
    )(x, s0, s1, d0, d1, w_self, w_s2d, w_d2s, b_self, b_s2d, b_d2s)


def kernel(x, edge_index, W_self, b_self, W_s2d, b_s2d, W_d2s, b_d2s):
    edge_flat = edge_index.reshape(2 * E)
    sums, degs = _sc_aggregate(x, edge_flat)
    d2 = degs.reshape(NUM_CORES, NPAD)
    return _finalize(x, sums[0], sums[1],
                     d2[0, :N].reshape(N, 1), d2[1, :N].reshape(N, 1),
                     W_self, W_s2d, W_d2s, b_self, b_s2d, b_d2s)


# speedup vs baseline: 11.4423x; 1.8306x over previous
import jax
import jax.numpy as jnp
from jax import lax
from jax.experimental import pallas as pl
from jax.experimental.pallas import tpu as pltpu
from jax.experimental.pallas import tpu_sc as plsc

N = 10000
E = 320000
D = 128
NUM_CORES = 2
NUM_SUBCORES = 16
EDGES_PER_TILE = E // NUM_SUBCORES
CHUNK = 128
FULL_CHUNKS = EDGES_PER_TILE // CHUNK
REM = EDGES_PER_TILE - FULL_CHUNKS * CHUNK
NPAD = 10240
STRIPE = NPAD // NUM_SUBCORES
C_S2D = 0.5
C_D2S = 0.5


def _sc_body(x_hbm, edge_hbm, s_out, deg_out,
             gidx0, sidx0, rows0, gidx1, sidx1, rows1, ones_v, dstage_v,
             gidx_r, sidx_r, rows_r, accum, degacc,
             gs0, gs1, is0, is1):
    c = lax.axis_index("c")
    s = lax.axis_index("s")
    g_base = c * E
    s_base = (1 - c) * E

    zero16 = jnp.zeros((16,), jnp.float32)
    one16 = jnp.ones((16,), jnp.float32)

    def _zero_row(i, carry):
        for j in range(D // 16):
            rows0[i, pl.ds(j * 16, 16)] = zero16
        return carry

    def _zero_dstage(i, carry):
        dstage_v[pl.ds(i * 16, 16)] = zero16
        return carry

    def _fill_ones(i, carry):
        ones_v[pl.ds(i * 16, 16)] = one16
        return carry

    lax.fori_loop(0, CHUNK, _zero_row, 0)
    lax.fori_loop(0, STRIPE // 16, _zero_dstage, 0)
    lax.fori_loop(0, CHUNK // 16, _fill_ones, 0)

    r0 = pl.multiple_of(s * STRIPE, 8)
    for k in range(STRIPE // CHUNK):
        pltpu.sync_copy(rows0, accum.at[pl.ds(r0 + k * CHUNK, CHUNK)])
    pltpu.sync_copy(dstage_v, degacc.at[pl.ds(r0, STRIPE)])
    plsc.subcore_barrier()

    base = s * EDGES_PER_TILE
    bufs = ((gidx0, sidx0, rows0, gs0, is0),
            (gidx1, sidx1, rows1, gs1, is1))

    def _g_slice(g):
        return edge_hbm.at[pl.ds(pl.multiple_of(g_base + base + g * CHUNK, 8),
                                 CHUNK)]

    def _s_slice(g):
        return edge_hbm.at[pl.ds(pl.multiple_of(s_base + base + g * CHUNK, 8),
                                 CHUNK)]

    def _idx_start(g, b):
        gidx, sidx, _, _, isem = bufs[b]
        pltpu.async_copy(_g_slice(g), gidx, isem)
        pltpu.async_copy(_s_slice(g), sidx, isem)

    def _idx_wait(b):
        gidx, sidx, _, _, isem = bufs[b]
        pltpu.make_async_copy(_g_slice(0), gidx, isem).wait()
        pltpu.make_async_copy(_s_slice(0), sidx, isem).wait()

    def _gather_start(b):
        gidx, _, rows, gsem, _ = bufs[b]
        pltpu.async_copy(x_hbm.at[gidx], rows, gsem)

    def _gather_wait(b):
        gidx, _, rows, gsem, _ = bufs[b]
        pltpu.make_async_copy(x_hbm.at[gidx], rows, gsem).wait()

    def _scatter(b):
        _, sidx, rows, _, _ = bufs[b]
        pltpu.sync_copy(rows, accum.at[sidx], add=True)
        pltpu.sync_copy(ones_v, degacc.at[sidx], add=True)

    pltpu.sync_copy(_g_slice(0), gidx0)
    pltpu.sync_copy(_s_slice(0), sidx0)
    _gather_start(0)
    _idx_start(1, 1)

    def _step(g, b):
        _gather_wait(b)
        _idx_wait(1 - b)
        _gather_start(1 - b)
        _scatter(b)
        _idx_start(g + 2, b)

    def _pair(i, carry):
        g = 2 * i
        _step(g, 0)
        _step(g + 1, 1)
        return carry

    lax.fori_loop(0, (FULL_CHUNKS - 2) // 2, _pair, 0)
    _gather_wait(0)
    _idx_wait(1)
    _gather_start(1)
    _scatter(0)
    _gather_wait(1)
    _scatter(1)

    offr = base + FULL_CHUNKS * CHUNK
    pltpu.sync_copy(edge_hbm.at[pl.ds(pl.multiple_of(g_base + offr, 8), REM)],
                    gidx_r)
    pltpu.sync_copy(edge_hbm.at[pl.ds(pl.multiple_of(s_base + offr, 8), REM)],
                    sidx_r)
    pltpu.async_copy(x_hbm.at[gidx_r], rows_r, gs0).wait()
    pltpu.sync_copy(rows_r, accum.at[sidx_r], add=True)
    pltpu.sync_copy(ones_v.at[pl.ds(0, REM)], degacc.at[sidx_r], add=True)

    plsc.subcore_barrier()

    for k in range(STRIPE // CHUNK):
        rk = pl.multiple_of(r0 + k * CHUNK, 8)
        pltpu.sync_copy(accum.at[pl.ds(rk, CHUNK)], rows0)
        pltpu.sync_copy(rows0, s_out.at[c, pl.ds(rk, CHUNK)])
    pltpu.sync_copy(degacc.at[pl.ds(r0, STRIPE)], dstage_v)
    pltpu.sync_copy(dstage_v,
                    deg_out.at[pl.ds(pl.multiple_of(c * NPAD + r0, 8),
                                     STRIPE)])


_sc_aggregate = pl.kernel(
    _sc_body,
    out_type=(
        jax.ShapeDtypeStruct((NUM_CORES, NPAD, D), jnp.float32),
        jax.ShapeDtypeStruct((NUM_CORES * NPAD,), jnp.float32),
    ),
    mesh=plsc.VectorSubcoreMesh(
        core_axis_name="c", subcore_axis_name="s",
        num_cores=NUM_CORES, num_subcores=NUM_SUBCORES),
    scratch_types=[
        pltpu.VMEM((CHUNK,), jnp.int32),
        pltpu.VMEM((CHUNK,), jnp.int32),
        pltpu.VMEM((CHUNK, D), jnp.float32),
        pltpu.VMEM((CHUNK,), jnp.int32),
        pltpu.VMEM((CHUNK,), jnp.int32),
        pltpu.VMEM((CHUNK, D), jnp.float32),
        pltpu.VMEM((CHUNK,), jnp.float32),
        pltpu.VMEM((STRIPE,), jnp.float32),
        pltpu.VMEM((REM,), jnp.int32),
        pltpu.VMEM((REM,), jnp.int32),
        pltpu.VMEM((REM, D), jnp.float32),
        pltpu.VMEM_SHARED((NPAD, D), jnp.float32),
        pltpu.VMEM_SHARED((NPAD,), jnp.float32),
        pltpu.SemaphoreType.DMA,
        pltpu.SemaphoreType.DMA,
        pltpu.SemaphoreType.DMA,
        pltpu.SemaphoreType.DMA,
    ],
)


BLK = 1000


def _fin_body(x_ref, s0_ref, s1_ref, d0_ref, d1_ref, ws_ref, w1_ref, w2_ref,
              bs_ref, b1_ref, b2_ref, o_ref):
    inv0 = C_S2D / jnp.maximum(d0_ref[...], 1.0)
    inv1 = C_D2S / jnp.maximum(d1_ref[...], 1.0)
    acc = jnp.dot(x_ref[...], ws_ref[...], preferred_element_type=jnp.float32)
    acc = acc + jnp.dot(s0_ref[...] * inv0, w1_ref[...],
                        preferred_element_type=jnp.float32)
    acc = acc + jnp.dot(s1_ref[...] * inv1, w2_ref[...],
                        preferred_element_type=jnp.float32)
    bias = bs_ref[...] + C_S2D * b1_ref[...] + C_D2S * b2_ref[...]
    o_ref[...] = acc + bias[None, :]


def _finalize(x, s0, s1, d0, d1, w_self, w_s2d, w_d2s, b_self, b_s2d, b_d2s):
    row_spec = pl.BlockSpec((BLK, D), lambda i: (i, 0))
    deg_spec = pl.BlockSpec((BLK, 1), lambda i: (i, 0))
    w_spec = pl.BlockSpec((D, D), lambda i: (0, 0))
    b_spec = pl.BlockSpec((D,), lambda i: (0,))
    return pl.pallas_call(
        _fin_body,
        grid=(N // BLK,),
        in_specs=[row_spec, row_spec, row_spec, deg_spec, deg_spec,
                  w_spec, w_spec, w_spec, b_spec, b_spec, b_spec],
        out_specs=row_spec,
        out_shape=jax.ShapeDtypeStruct((N, D), jnp.float32),
    )(x, s0, s1, d0, d1, w_self, w_s2d, w_d2s, b_self, b_s2d, b_d2s)


def kernel(x, edge_index, W_self, b_self, W_s2d, b_s2d, W_d2s, b_d2s):
    edge_flat = edge_index.reshape(2 * E)
    sums, degs = _sc_aggregate(x, edge_flat)
    d2 = degs.reshape(NUM_CORES, NPAD)
    return _finalize(x, sums[0], sums[1],
                     d2[0, :N].reshape(N, 1), d2[1, :N].reshape(N, 1),
                     W_self, W_s2d, W_d2s, b_self, b_s2d, b_d2s)
